# baseline (speedup 1.0000x reference)
import functools

import jax
import jax.numpy as jnp
from jax import lax
from jax.experimental import pallas as pl
from jax.experimental.pallas import tpu as pltpu
from jax.experimental.pallas import tpu_sc as plsc

_INFO = plsc.get_sparse_core_info()
_NC = _INFO.num_cores
_NS = _INFO.num_subcores
_NW = _NC * _NS


def _sc_gather(table, idx0, idx1, seq_len):
    B = idx0.shape[0]
    D = table.shape[1]
    b_per_w = B // _NW
    mesh = plsc.VectorSubcoreMesh(core_axis_name="c", subcore_axis_name="s")

    @functools.partial(
        pl.kernel,
        mesh=mesh,
        out_type=jax.ShapeDtypeStruct((B, D), jnp.float32),
        scratch_types=[
            pltpu.VMEM((b_per_w,), jnp.int32),
            pltpu.VMEM((b_per_w,), jnp.int32),
            pltpu.VMEM((b_per_w, D), jnp.float32),
            pltpu.SemaphoreType.DMA,
        ],
    )
    def gather_kernel(table_hbm, i0_hbm, i1_hbm, out_hbm, i0_v, i1_v, rows_v, sem):
        wid = lax.axis_index("s") * _NC + lax.axis_index("c")
        base = wid * b_per_w
        pltpu.sync_copy(i0_hbm.at[pl.ds(base, b_per_w)], i0_v)
        pltpu.sync_copy(i1_hbm.at[pl.ds(base, b_per_w)], i1_v)
        i0_v[...] = i0_v[...] * seq_len + i1_v[...]
        pltpu.async_copy(table_hbm.at[i0_v], rows_v, sem).wait()
        pltpu.sync_copy(rows_v, out_hbm.at[pl.ds(base, b_per_w)])

    return gather_kernel(table, idx0, idx1)


def _tc_head(x, W, b2d):
    B, D = x.shape

    def body(x_ref, w_ref, b_ref, o_ref):
        acc = lax.dot_general(
            x_ref[...], w_ref[...],
            (((1,), (1,)), ((), ())),
            preferred_element_type=jnp.float32,
        )
        o_ref[...] = jnp.tanh(acc + b_ref[...])

    return pl.pallas_call(
        body,
        out_shape=jax.ShapeDtypeStruct((B, D), jnp.float32),
    )(x, W, b2d)


def kernel(hidden_states, cls_indexes, W, b):
    n_batch, seq_len, D = hidden_states.shape
    table = hidden_states.reshape(n_batch * seq_len, D)
    idx = cls_indexes.astype(jnp.int32)
    x = _sc_gather(table, idx[:, 0], idx[:, 1], seq_len)
    return _tc_head(x, W, b.reshape(1, D))
